# baseline (speedup 1.0000x reference)
import functools

import jax
import jax.numpy as jnp
from jax import lax
from jax.experimental import pallas as pl
from jax.experimental.pallas import tpu as pltpu
from jax.experimental.pallas import tpu_sc as plsc

N_B = 10000
N_E = 160000
N_A = 5000
NODE_DIM = 128
EDGE_DIM = 128
ATOM_DIM = 64
HID = 64

NC = 2
NS = 16
NWK = NC * NS
E_PAD = 163840
GCHUNK = 128
EPW_A = 10240
EPW_B = 0
GITER_A = EPW_A // GCHUNK
SPW = N_E // NWK
SCHUNK = 128
SITER = SPW // SCHUNK
STAIL = SPW - SITER * SCHUNK
SPAIRS = (SITER - 1) // 2
NB_PAD = 10240
RPT = NB_PAD // NS
ZROWS = 64


def _node_tables_body(nf_ref, nw_ref, wsrc_ref, wdst_ref, ts_ref, td_ref):
    x = nf_ref[...]
    proj = jnp.dot(x, wsrc_ref[...], preferred_element_type=jnp.float32)
    pi = lax.bitcast_convert_type(proj, jnp.uint32) + jnp.uint32(0x8000)
    ni = lax.bitcast_convert_type(nw_ref[...], jnp.uint32) + jnp.uint32(0x8000)
    word = (ni & jnp.uint32(0xFFFF0000)) | (pi >> 16)
    ts_ref[...] = lax.bitcast_convert_type(word, jnp.float32)
    td_ref[...] = jnp.dot(x, wdst_ref[...], preferred_element_type=jnp.float32)


def _vert_table_body(af_ref, wv_ref, tv_ref):
    tv_ref[...] = jnp.dot(af_ref[...], wv_ref[...], preferred_element_type=jnp.float32)


def _edge_mlp_body(gnw_ref, gdv_ref, ef_ref, we_ref, be_ref, wd_ref, bd_ref, m_ref):
    eproj = (jnp.dot(ef_ref[...], we_ref[...], preferred_element_type=jnp.float32)
             + be_ref[...])
    wi = lax.bitcast_convert_type(gnw_ref[...], jnp.uint32)
    sproj = lax.bitcast_convert_type(wi << 16, jnp.float32)
    nw = lax.bitcast_convert_type(wi & jnp.uint32(0xFFFF0000), jnp.float32)
    pre = sproj + gdv_ref[...] + eproj
    h1 = pre * jax.nn.sigmoid(pre)
    z = jnp.dot(h1, wd_ref[...], preferred_element_type=jnp.float32) + bd_ref[...]
    gate = jax.nn.sigmoid(z[:, :EDGE_DIM])
    zo = z[:, EDGE_DIM:]
    outp = zo * jax.nn.sigmoid(zo)
    m_ref[...] = outp * gate * nw


def _final_body(p_ref, nw_ref, nf_ref, wl_ref, bl_ref, o_ref):
    h = (p_ref[0] + p_ref[1]) * nw_ref[...]
    o_ref[...] = (nf_ref[...]
                  + jnp.dot(h, wl_ref[...], preferred_element_type=jnp.float32)
                  + bl_ref[...])


_MESH = plsc.VectorSubcoreMesh(core_axis_name="c", subcore_axis_name="s",
                               num_cores=NC, num_subcores=NS)

_GSCRATCH = [
    pltpu.VMEM((EPW_A,), jnp.int32),
    pltpu.VMEM((EPW_A,), jnp.int32),
    pltpu.VMEM((EPW_A,), jnp.int32),
]
for _ in range(2):
    _GSCRATCH += [
        pltpu.VMEM((GCHUNK, 128), jnp.float32),
        pltpu.VMEM((GCHUNK, 128), jnp.float32),
        pltpu.VMEM((GCHUNK, 128), jnp.float32),
        pltpu.SemaphoreType.DMA,
        pltpu.SemaphoreType.DMA,
    ]


@functools.partial(
    pl.kernel,
    out_type=(jax.ShapeDtypeStruct((E_PAD, 128), jnp.float32),
              jax.ShapeDtypeStruct((E_PAD, 128), jnp.float32)),
    mesh=_MESH,
    scratch_types=_GSCRATCH,
)
def _sc_gather(src_h, dst_h, vid_h, tsrc_h, tdst_h, tvert_h,
               gnw_out, gdv_out, sidx, didx, vidx,
               snw0, drows0, vrows0, gsem0, ssem0,
               snw1, drows1, vrows1, gsem1, ssem1):
    c = lax.axis_index("c")
    s = lax.axis_index("s")
    sets = ((snw0, drows0, vrows0, gsem0, ssem0),
            (snw1, drows1, vrows1, gsem1, ssem1))

    def run(base, epw, giter):
        pltpu.sync_copy(src_h.at[pl.ds(base, epw)], sidx.at[pl.ds(0, epw)])
        pltpu.sync_copy(dst_h.at[pl.ds(base, epw)], didx.at[pl.ds(0, epw)])
        pltpu.sync_copy(vid_h.at[pl.ds(base, epw)], vidx.at[pl.ds(0, epw)])

        def gfire(bufs, i):
            snw, drows, vrows, gsem, ssem = bufs
            off = pl.multiple_of(i * GCHUNK, 8)
            pltpu.async_copy(tsrc_h.at[sidx.at[pl.ds(off, GCHUNK)]], snw, gsem)
            pltpu.async_copy(tdst_h.at[didx.at[pl.ds(off, GCHUNK)]], drows, gsem)
            pltpu.async_copy(tvert_h.at[vidx.at[pl.ds(off, GCHUNK)]], vrows, gsem)

        def consume(bufs, i):
            snw, drows, vrows, gsem, ssem = bufs
            cb = base + i * GCHUNK
            pltpu.make_async_copy(tsrc_h.at[sidx.at[pl.ds(0, GCHUNK)]], snw, gsem).wait()
            pltpu.make_async_copy(tsrc_h.at[sidx.at[pl.ds(0, GCHUNK)]], drows, gsem).wait()
            pltpu.make_async_copy(tsrc_h.at[sidx.at[pl.ds(0, GCHUNK)]], vrows, gsem).wait()
            pltpu.async_copy(snw, gnw_out.at[pl.ds(cb, GCHUNK)], ssem)

            def edge_body(e, cin):
                for k in range(8):
                    sl = pl.ds(k * 16, 16)
                    drows[e, sl] = drows[e, sl] + vrows[e, sl]
                return cin

            lax.fori_loop(0, GCHUNK, edge_body, 0)
            pltpu.async_copy(drows, gdv_out.at[pl.ds(cb, GCHUNK)], ssem)

        def swait(bufs):
            snw, drows, vrows, gsem, ssem = bufs
            pltpu.make_async_copy(snw, gnw_out.at[pl.ds(0, GCHUNK)], ssem).wait()
            pltpu.make_async_copy(drows, gdv_out.at[pl.ds(0, GCHUNK)], ssem).wait()

        gfire(sets[0], 0)
        gfire(sets[1], 1)

        def step(g, carry):
            i = 2 * g
            consume(sets[0], i)
            consume(sets[1], i + 1)
            swait(sets[0])
            gfire(sets[0], i + 2)
            swait(sets[1])
            gfire(sets[1], i + 3)
            return carry

        lax.fori_loop(0, giter // 2 - 1, step, 0)
        consume(sets[0], giter - 2)
        consume(sets[1], giter - 1)
        swait(sets[0])
        swait(sets[1])

    @pl.when(c == 0)
    def _():
        run(s * EPW_A, EPW_A, GITER_A)


_SSCRATCH = []
for _ in range(2):
    _SSCRATCH += [
        pltpu.VMEM((SCHUNK,), jnp.int32),
        pltpu.VMEM((SCHUNK, 128), jnp.float32),
        pltpu.SemaphoreType.DMA,
    ]
_SSCRATCH += [
    pltpu.VMEM((STAIL,), jnp.int32),
    pltpu.VMEM((STAIL, 128), jnp.float32),
    pltpu.VMEM((ZROWS, 128), jnp.float32),
    pltpu.VMEM_SHARED((NB_PAD, 128), jnp.float32),
]


@functools.partial(
    pl.kernel,
    out_type=jax.ShapeDtypeStruct((NC, NB_PAD, 128), jnp.float32),
    mesh=_MESH,
    scratch_types=_SSCRATCH,
)
def _sc_scatter(m_h, dst_h, part_out,
                didx0, mrows0, sem0, didx1, mrows1, sem1,
                tidx, trows, zbuf, acc):
    c = lax.axis_index("c")
    s = lax.axis_index("s")
    sets = ((didx0, mrows0, sem0), (didx1, mrows1, sem1))

    def zrow(e, carry):
        for k in range(8):
            zbuf[e, pl.ds(k * 16, 16)] = jnp.zeros((16,), jnp.float32)
        return carry

    lax.fori_loop(0, ZROWS, zrow, 0)
    for j in range(RPT // ZROWS):
        pltpu.sync_copy(zbuf, acc.at[pl.ds(s * RPT + j * ZROWS, ZROWS)])
    plsc.subcore_barrier()

    base = (c * NS + s) * SPW

    def fire(bufs, cb):
        didx, mrows, sem = bufs
        pltpu.async_copy(dst_h.at[pl.ds(cb, SCHUNK)], didx, sem)
        pltpu.async_copy(m_h.at[pl.ds(cb, SCHUNK)], mrows, sem)

    def consume(bufs):
        didx, mrows, sem = bufs
        pltpu.make_async_copy(dst_h.at[pl.ds(0, SCHUNK)], didx, sem).wait()
        pltpu.make_async_copy(m_h.at[pl.ds(0, SCHUNK)], mrows, sem).wait()
        pltpu.sync_copy(mrows, acc.at[didx], add=True)

    fire(sets[0], base)

    def step(g, carry):
        cb = base + 2 * g * SCHUNK
        fire(sets[1], cb + SCHUNK)
        consume(sets[0])
        fire(sets[0], cb + 2 * SCHUNK)
        consume(sets[1])
        return carry

    lax.fori_loop(0, SPAIRS, step, 0)
    consume(sets[0])

    tb = base + SITER * SCHUNK
    pltpu.sync_copy(dst_h.at[pl.ds(tb, STAIL)], tidx)
    pltpu.sync_copy(m_h.at[pl.ds(tb, STAIL)], trows)
    pltpu.sync_copy(trows, acc.at[tidx], add=True)
    plsc.subcore_barrier()
    pltpu.sync_copy(acc.at[pl.ds(s * RPT, RPT)], part_out.at[c, pl.ds(s * RPT, RPT)])


def kernel(graph_edges, node_feat, edge_feat, node_weight, edge_index, atom_feat,
           gw_W1, gw_b1, gw_W2, gw_b2, out_W1, out_b1, out_W2, out_b2, lin_W, lin_b):
    f32 = jnp.float32
    src = graph_edges[0].astype(jnp.int32)
    dst = graph_edges[1].astype(jnp.int32)
    vid = edge_index[:, 1].astype(jnp.int32)
    pad = E_PAD - N_E
    src_p = jnp.pad(src, (0, pad))
    dst_p = jnp.pad(dst, (0, pad))
    vid_p = jnp.pad(vid, (0, pad))

    w_src = jnp.concatenate([gw_W1[0:128], out_W1[0:128]], axis=1)
    w_dst = jnp.concatenate([gw_W1[128:256], out_W1[128:256]], axis=1)
    w_edge = jnp.concatenate([gw_W1[256:384], out_W1[256:384]], axis=1)
    w_vert = jnp.concatenate([gw_W1[384:448], out_W1[384:448]], axis=1)
    b_edge = jnp.concatenate([gw_b1, out_b1]).reshape(1, 128)
    w_diag = jnp.zeros((128, 256), f32)
    w_diag = w_diag.at[0:64, 0:128].set(gw_W2).at[64:128, 128:256].set(out_W2)
    b_diag = jnp.concatenate([gw_b2, out_b2]).reshape(1, 256)
    b_lin = lin_b.reshape(1, 128)

    nb_blk = 2000
    t_srcnw, t_dst = pl.pallas_call(
        _node_tables_body,
        grid=(N_B // nb_blk,),
        in_specs=[pl.BlockSpec((nb_blk, 128), lambda i: (i, 0)),
                  pl.BlockSpec((nb_blk, 128), lambda i: (i, 0)),
                  pl.BlockSpec((128, 128), lambda i: (0, 0)),
                  pl.BlockSpec((128, 128), lambda i: (0, 0))],
        out_specs=[pl.BlockSpec((nb_blk, 128), lambda i: (i, 0)),
                   pl.BlockSpec((nb_blk, 128), lambda i: (i, 0))],
        out_shape=[jax.ShapeDtypeStruct((N_B, 128), f32),
                   jax.ShapeDtypeStruct((N_B, 128), f32)],
    )(node_feat, node_weight, w_src, w_dst)

    t_vert = pl.pallas_call(
        _vert_table_body,
        grid=(1,),
        in_specs=[pl.BlockSpec((N_A, ATOM_DIM), lambda i: (0, 0)),
                  pl.BlockSpec((ATOM_DIM, 128), lambda i: (0, 0))],
        out_specs=pl.BlockSpec((N_A, 128), lambda i: (0, 0)),
        out_shape=jax.ShapeDtypeStruct((N_A, 128), f32),
    )(atom_feat, w_vert)

    gnw, gdv = _sc_gather(src_p, dst_p, vid_p, t_srcnw, t_dst, t_vert)

    ne_blk = 2000
    m = pl.pallas_call(
        _edge_mlp_body,
        grid=(N_E // ne_blk,),
        in_specs=[pl.BlockSpec((ne_blk, 128), lambda i: (i, 0)),
                  pl.BlockSpec((ne_blk, 128), lambda i: (i, 0)),
                  pl.BlockSpec((ne_blk, 128), lambda i: (i, 0)),
                  pl.BlockSpec((128, 128), lambda i: (0, 0)),
                  pl.BlockSpec((1, 128), lambda i: (0, 0)),
                  pl.BlockSpec((128, 256), lambda i: (0, 0)),
                  pl.BlockSpec((1, 256), lambda i: (0, 0))],
        out_specs=pl.BlockSpec((ne_blk, 128), lambda i: (i, 0)),
        out_shape=jax.ShapeDtypeStruct((N_E, 128), f32),
    )(gnw, gdv, edge_feat, w_edge, b_edge, w_diag, b_diag)

    partials = _sc_scatter(m, dst)

    out = pl.pallas_call(
        _final_body,
        grid=(N_B // nb_blk,),
        in_specs=[pl.BlockSpec((NC, nb_blk, 128), lambda i: (0, i, 0)),
                  pl.BlockSpec((nb_blk, 128), lambda i: (i, 0)),
                  pl.BlockSpec((nb_blk, 128), lambda i: (i, 0)),
                  pl.BlockSpec((128, 128), lambda i: (0, 0)),
                  pl.BlockSpec((1, 128), lambda i: (0, 0))],
        out_specs=pl.BlockSpec((nb_blk, 128), lambda i: (i, 0)),
        out_shape=jax.ShapeDtypeStruct((N_B, 128), f32),
    )(partials, node_weight, node_feat, lin_W, b_lin)

    return out


# speedup vs baseline: 1.1998x; 1.1998x over previous
import functools

import jax
import jax.numpy as jnp
from jax import lax
from jax.experimental import pallas as pl
from jax.experimental.pallas import tpu as pltpu
from jax.experimental.pallas import tpu_sc as plsc

N_B = 10000
N_E = 160000
N_A = 5000
NODE_DIM = 128
EDGE_DIM = 128
ATOM_DIM = 64
HID = 64

NC = 2
NS = 16
NWK = NC * NS
E_PAD = 163840
GCHUNK = 128
E_HALF = 80000
EH_PAD = 81920
EPW_A = 3584
EPW_B = 1536
GITER_A = EPW_A // GCHUNK
GITER_B = EPW_B // GCHUNK
SPW = N_E // NWK
SCHUNK = 128
SITER = SPW // SCHUNK
STAIL = SPW - SITER * SCHUNK
SPAIRS = (SITER - 1) // 2
NB_PAD = 10240
RPT = NB_PAD // NS
ZROWS = 64


def _node_tables_body(nf_ref, nw_ref, wsrc_ref, wdst_ref, ts_ref, td_ref):
    x = nf_ref[...]
    proj = jnp.dot(x, wsrc_ref[...], preferred_element_type=jnp.float32)
    pi = lax.bitcast_convert_type(proj, jnp.uint32) + jnp.uint32(0x8000)
    ni = lax.bitcast_convert_type(nw_ref[...], jnp.uint32) + jnp.uint32(0x8000)
    word = (ni & jnp.uint32(0xFFFF0000)) | (pi >> 16)
    ts_ref[...] = lax.bitcast_convert_type(word, jnp.float32)
    td_ref[...] = jnp.dot(x, wdst_ref[...], preferred_element_type=jnp.float32)


def _vert_table_body(af_ref, wv_ref, tv_ref):
    tv_ref[...] = jnp.dot(af_ref[...], wv_ref[...], preferred_element_type=jnp.float32)


def _edge_mlp_body(gnw_ref, gdv_ref, ef_ref, we_ref, be_ref, wd_ref, bd_ref, m_ref):
    eproj = (jnp.dot(ef_ref[...], we_ref[...], preferred_element_type=jnp.float32)
             + be_ref[...])
    wi = lax.bitcast_convert_type(gnw_ref[...], jnp.uint32)
    sproj = lax.bitcast_convert_type(wi << 16, jnp.float32)
    nw = lax.bitcast_convert_type(wi & jnp.uint32(0xFFFF0000), jnp.float32)
    pre = sproj + gdv_ref[...] + eproj
    h1 = pre * jax.nn.sigmoid(pre)
    z = jnp.dot(h1, wd_ref[...], preferred_element_type=jnp.float32) + bd_ref[...]
    gate = jax.nn.sigmoid(z[:, :EDGE_DIM])
    zo = z[:, EDGE_DIM:]
    outp = zo * jax.nn.sigmoid(zo)
    m_ref[...] = outp * gate * nw


def _final_body(p_ref, nw_ref, nf_ref, wl_ref, bl_ref, o_ref):
    h = (p_ref[0] + p_ref[1]) * nw_ref[...]
    o_ref[...] = (nf_ref[...]
                  + jnp.dot(h, wl_ref[...], preferred_element_type=jnp.float32)
                  + bl_ref[...])


_MESH = plsc.VectorSubcoreMesh(core_axis_name="c", subcore_axis_name="s",
                               num_cores=NC, num_subcores=NS)

_GSCRATCH = [
    pltpu.VMEM((EPW_A,), jnp.int32),
    pltpu.VMEM((EPW_A,), jnp.int32),
    pltpu.VMEM((EPW_A,), jnp.int32),
]
for _ in range(2):
    _GSCRATCH += [
        pltpu.VMEM((GCHUNK, 128), jnp.float32),
        pltpu.VMEM((GCHUNK, 128), jnp.float32),
        pltpu.VMEM((GCHUNK, 128), jnp.float32),
        pltpu.SemaphoreType.DMA,
        pltpu.SemaphoreType.DMA,
    ]


@functools.partial(
    pl.kernel,
    out_type=(jax.ShapeDtypeStruct((EH_PAD, 128), jnp.float32),
              jax.ShapeDtypeStruct((EH_PAD, 128), jnp.float32)),
    mesh=_MESH,
    scratch_types=_GSCRATCH,
)
def _sc_gather(src_h, dst_h, vid_h, tsrc_h, tdst_h, tvert_h,
               gnw_out, gdv_out, sidx, didx, vidx,
               snw0, drows0, vrows0, gsem0, ssem0,
               snw1, drows1, vrows1, gsem1, ssem1):
    c = lax.axis_index("c")
    s = lax.axis_index("s")
    sets = ((snw0, drows0, vrows0, gsem0, ssem0),
            (snw1, drows1, vrows1, gsem1, ssem1))

    def run(base, epw, giter):
        pltpu.sync_copy(src_h.at[pl.ds(base, epw)], sidx.at[pl.ds(0, epw)])
        pltpu.sync_copy(dst_h.at[pl.ds(base, epw)], didx.at[pl.ds(0, epw)])
        pltpu.sync_copy(vid_h.at[pl.ds(base, epw)], vidx.at[pl.ds(0, epw)])

        def gfire(bufs, i):
            snw, drows, vrows, gsem, ssem = bufs
            off = pl.multiple_of(i * GCHUNK, 8)
            pltpu.async_copy(tsrc_h.at[sidx.at[pl.ds(off, GCHUNK)]], snw, gsem)
            pltpu.async_copy(tdst_h.at[didx.at[pl.ds(off, GCHUNK)]], drows, gsem)
            pltpu.async_copy(tvert_h.at[vidx.at[pl.ds(off, GCHUNK)]], vrows, gsem)

        def consume(bufs, i):
            snw, drows, vrows, gsem, ssem = bufs
            cb = base + i * GCHUNK
            pltpu.make_async_copy(tsrc_h.at[sidx.at[pl.ds(0, GCHUNK)]], snw, gsem).wait()
            pltpu.make_async_copy(tsrc_h.at[sidx.at[pl.ds(0, GCHUNK)]], drows, gsem).wait()
            pltpu.make_async_copy(tsrc_h.at[sidx.at[pl.ds(0, GCHUNK)]], vrows, gsem).wait()
            pltpu.async_copy(snw, gnw_out.at[pl.ds(cb, GCHUNK)], ssem)

            def edge_body(e, cin):
                for k in range(8):
                    sl = pl.ds(k * 16, 16)
                    drows[e, sl] = drows[e, sl] + vrows[e, sl]
                return cin

            lax.fori_loop(0, GCHUNK, edge_body, 0)
            pltpu.async_copy(drows, gdv_out.at[pl.ds(cb, GCHUNK)], ssem)

        def swait(bufs):
            snw, drows, vrows, gsem, ssem = bufs
            pltpu.make_async_copy(snw, gnw_out.at[pl.ds(0, GCHUNK)], ssem).wait()
            pltpu.make_async_copy(drows, gdv_out.at[pl.ds(0, GCHUNK)], ssem).wait()

        gfire(sets[0], 0)
        gfire(sets[1], 1)

        def step(g, carry):
            i = 2 * g
            consume(sets[0], i)
            consume(sets[1], i + 1)
            swait(sets[0])
            gfire(sets[0], i + 2)
            swait(sets[1])
            gfire(sets[1], i + 3)
            return carry

        lax.fori_loop(0, giter // 2 - 1, step, 0)
        consume(sets[0], giter - 2)
        consume(sets[1], giter - 1)
        swait(sets[0])
        swait(sets[1])

    @pl.when(c == 0)
    def _():
        run(s * EPW_A, EPW_A, GITER_A)

    @pl.when(c == 1)
    def _():
        run(NS * EPW_A + s * EPW_B, EPW_B, GITER_B)


_SSCRATCH = []
for _ in range(2):
    _SSCRATCH += [
        pltpu.VMEM((SCHUNK,), jnp.int32),
        pltpu.VMEM((SCHUNK, 128), jnp.float32),
        pltpu.SemaphoreType.DMA,
    ]
_SSCRATCH += [
    pltpu.VMEM((STAIL,), jnp.int32),
    pltpu.VMEM((STAIL, 128), jnp.float32),
    pltpu.VMEM((ZROWS, 128), jnp.float32),
    pltpu.VMEM_SHARED((NB_PAD, 128), jnp.float32),
]


@functools.partial(
    pl.kernel,
    out_type=jax.ShapeDtypeStruct((NC, NB_PAD, 128), jnp.float32),
    mesh=_MESH,
    scratch_types=_SSCRATCH,
)
def _sc_scatter(m1_h, m2_h, dst_h, part_out,
                didx0, mrows0, sem0, didx1, mrows1, sem1,
                tidx, trows, zbuf, acc):
    c = lax.axis_index("c")
    s = lax.axis_index("s")
    sets = ((didx0, mrows0, sem0), (didx1, mrows1, sem1))

    def zrow(e, carry):
        for k in range(8):
            zbuf[e, pl.ds(k * 16, 16)] = jnp.zeros((16,), jnp.float32)
        return carry

    lax.fori_loop(0, ZROWS, zrow, 0)
    for j in range(RPT // ZROWS):
        pltpu.sync_copy(zbuf, acc.at[pl.ds(s * RPT + j * ZROWS, ZROWS)])
    plsc.subcore_barrier()

    gbase = (c * NS + s) * SPW
    mbase = s * SPW

    def run(m_h):
        def fire(bufs, i):
            didx, mrows, sem = bufs
            pltpu.async_copy(dst_h.at[pl.ds(gbase + i * SCHUNK, SCHUNK)], didx, sem)
            pltpu.async_copy(m_h.at[pl.ds(mbase + i * SCHUNK, SCHUNK)], mrows, sem)

        def consume(bufs):
            didx, mrows, sem = bufs
            pltpu.make_async_copy(dst_h.at[pl.ds(0, SCHUNK)], didx, sem).wait()
            pltpu.make_async_copy(m_h.at[pl.ds(0, SCHUNK)], mrows, sem).wait()
            pltpu.sync_copy(mrows, acc.at[didx], add=True)

        fire(sets[0], 0)

        def step(g, carry):
            i = 2 * g
            fire(sets[1], i + 1)
            consume(sets[0])
            fire(sets[0], i + 2)
            consume(sets[1])
            return carry

        lax.fori_loop(0, SPAIRS, step, 0)
        consume(sets[0])

        tc_ = SITER * SCHUNK
        pltpu.sync_copy(dst_h.at[pl.ds(gbase + tc_, STAIL)], tidx)
        pltpu.sync_copy(m_h.at[pl.ds(mbase + tc_, STAIL)], trows)
        pltpu.sync_copy(trows, acc.at[tidx], add=True)

    @pl.when(c == 0)
    def _():
        run(m1_h)

    @pl.when(c == 1)
    def _():
        run(m2_h)

    plsc.subcore_barrier()
    pltpu.sync_copy(acc.at[pl.ds(s * RPT, RPT)], part_out.at[c, pl.ds(s * RPT, RPT)])


def kernel(graph_edges, node_feat, edge_feat, node_weight, edge_index, atom_feat,
           gw_W1, gw_b1, gw_W2, gw_b2, out_W1, out_b1, out_W2, out_b2, lin_W, lin_b):
    f32 = jnp.float32
    src = graph_edges[0].astype(jnp.int32)
    dst = graph_edges[1].astype(jnp.int32)
    vid = edge_index[:, 1].astype(jnp.int32)
    hpad = EH_PAD - E_HALF
    halves = []
    for lo in (0, E_HALF):
        halves.append(tuple(jnp.pad(a[lo:lo + E_HALF], (0, hpad))
                            for a in (src, dst, vid)))

    w_src = jnp.concatenate([gw_W1[0:128], out_W1[0:128]], axis=1)
    w_dst = jnp.concatenate([gw_W1[128:256], out_W1[128:256]], axis=1)
    w_edge = jnp.concatenate([gw_W1[256:384], out_W1[256:384]], axis=1)
    w_vert = jnp.concatenate([gw_W1[384:448], out_W1[384:448]], axis=1)
    b_edge = jnp.concatenate([gw_b1, out_b1]).reshape(1, 128)
    w_diag = jnp.zeros((128, 256), f32)
    w_diag = w_diag.at[0:64, 0:128].set(gw_W2).at[64:128, 128:256].set(out_W2)
    b_diag = jnp.concatenate([gw_b2, out_b2]).reshape(1, 256)
    b_lin = lin_b.reshape(1, 128)

    nb_blk = 2000
    t_srcnw, t_dst = pl.pallas_call(
        _node_tables_body,
        grid=(N_B // nb_blk,),
        in_specs=[pl.BlockSpec((nb_blk, 128), lambda i: (i, 0)),
                  pl.BlockSpec((nb_blk, 128), lambda i: (i, 0)),
                  pl.BlockSpec((128, 128), lambda i: (0, 0)),
                  pl.BlockSpec((128, 128), lambda i: (0, 0))],
        out_specs=[pl.BlockSpec((nb_blk, 128), lambda i: (i, 0)),
                   pl.BlockSpec((nb_blk, 128), lambda i: (i, 0))],
        out_shape=[jax.ShapeDtypeStruct((N_B, 128), f32),
                   jax.ShapeDtypeStruct((N_B, 128), f32)],
    )(node_feat, node_weight, w_src, w_dst)

    t_vert = pl.pallas_call(
        _vert_table_body,
        grid=(1,),
        in_specs=[pl.BlockSpec((N_A, ATOM_DIM), lambda i: (0, 0)),
                  pl.BlockSpec((ATOM_DIM, 128), lambda i: (0, 0))],
        out_specs=pl.BlockSpec((N_A, 128), lambda i: (0, 0)),
        out_shape=jax.ShapeDtypeStruct((N_A, 128), f32),
    )(atom_feat, w_vert)

    ne_blk = 2000

    def half_mlp(gnw, gdv, ef):
        return pl.pallas_call(
            _edge_mlp_body,
            grid=(E_HALF // ne_blk,),
            in_specs=[pl.BlockSpec((ne_blk, 128), lambda i: (i, 0)),
                      pl.BlockSpec((ne_blk, 128), lambda i: (i, 0)),
                      pl.BlockSpec((ne_blk, 128), lambda i: (i, 0)),
                      pl.BlockSpec((128, 128), lambda i: (0, 0)),
                      pl.BlockSpec((1, 128), lambda i: (0, 0)),
                      pl.BlockSpec((128, 256), lambda i: (0, 0)),
                      pl.BlockSpec((1, 256), lambda i: (0, 0))],
            out_specs=pl.BlockSpec((ne_blk, 128), lambda i: (i, 0)),
            out_shape=jax.ShapeDtypeStruct((E_HALF, 128), f32),
        )(gnw, gdv, ef, w_edge, b_edge, w_diag, b_diag)

    gnw1, gdv1 = _sc_gather(*halves[0], t_srcnw, t_dst, t_vert)
    gnw2, gdv2 = _sc_gather(*halves[1], t_srcnw, t_dst, t_vert)
    m1 = half_mlp(gnw1, gdv1, edge_feat[:E_HALF])
    m2 = half_mlp(gnw2, gdv2, edge_feat[E_HALF:])

    partials = _sc_scatter(m1, m2, dst)

    out = pl.pallas_call(
        _final_body,
        grid=(N_B // nb_blk,),
        in_specs=[pl.BlockSpec((NC, nb_blk, 128), lambda i: (0, i, 0)),
                  pl.BlockSpec((nb_blk, 128), lambda i: (i, 0)),
                  pl.BlockSpec((nb_blk, 128), lambda i: (i, 0)),
                  pl.BlockSpec((128, 128), lambda i: (0, 0)),
                  pl.BlockSpec((1, 128), lambda i: (0, 0))],
        out_specs=pl.BlockSpec((nb_blk, 128), lambda i: (i, 0)),
        out_shape=jax.ShapeDtypeStruct((N_B, 128), f32),
    )(partials, node_weight, node_feat, lin_W, b_lin)

    return out
